# baseline (speedup 1.0000x reference)
import functools

import jax
import jax.numpy as jnp
from jax.experimental import pallas as pl
from jax.experimental.pallas import tpu as pltpu

N_HEADS = 4
HEAD_DIM = 32
NEG_SLOPE = 0.2


def _msgs_kernel(ea_ref, w1_ref, c0_ref, w2_ref, b2_ref, out_ref):
    h = jnp.dot(ea_ref[...], w1_ref[...], preferred_element_type=jnp.float32)
    h = jnp.maximum(h + c0_ref[...], 0.0)
    out_ref[...] = (
        jnp.dot(h, w2_ref[...], preferred_element_type=jnp.float32) + b2_ref[...]
    )


def _node0_kernel(agg_ref, cnt_ref, wihT_ref, bih_ref, bhh_ref, gw_ref, a_ref,
                  xp_ref, att_ref):
    gi = jnp.dot(agg_ref[...], wihT_ref[...], preferred_element_type=jnp.float32)
    gi = gi + bih_ref[...]
    bhh = bhh_ref[...]
    r = jax.nn.sigmoid(gi[:, :128] + bhh[:, :128])
    z = jax.nn.sigmoid(gi[:, 128:256] + bhh[:, 128:256])
    cand = jnp.tanh(gi[:, 256:] + r * bhh[:, 256:])
    mem = (1.0 - z) * cand
    mem = jnp.where(cnt_ref[...] > 0.0, mem, 0.0)
    xp = jnp.dot(mem, gw_ref[...], preferred_element_type=jnp.float32)
    xp_ref[...] = xp
    att_ref[...] = jnp.dot(xp, a_ref[...], preferred_element_type=jnp.float32)


def _node1_kernel(out0_ref, gb_ref, gw_ref, a_ref, xp_ref, att_ref):
    v = out0_ref[...] + gb_ref[...]
    h = jnp.where(v > 0.0, v, jnp.exp(v) - 1.0)
    xp = jnp.dot(h, gw_ref[...], preferred_element_type=jnp.float32)
    xp_ref[...] = xp
    att_ref[...] = jnp.dot(xp, a_ref[...], preferred_element_type=jnp.float32)


def _pool_kernel(out1_ref, gb_ref, wc1_ref, bc1_ref, wc2_ref, bc2_ref, inv_ref,
                 res_ref, acc_ref):
    i = pl.program_id(0)

    @pl.when(i == 0)
    def _init():
        acc_ref[...] = jnp.zeros_like(acc_ref)

    v = out1_ref[...] + gb_ref[...]
    h = jnp.where(v > 0.0, v, jnp.exp(v) - 1.0)
    acc_ref[0:1, :] += jnp.sum(h, axis=0, keepdims=True)

    @pl.when(i == pl.num_programs(0) - 1)
    def _final():
        pooled = acc_ref[0:1, :] * inv_ref[...]
        t = jnp.maximum(
            jnp.dot(pooled, wc1_ref[...], preferred_element_type=jnp.float32)
            + bc1_ref[...],
            0.0,
        )
        res_ref[...] = (
            jnp.dot(t, wc2_ref[...], preferred_element_type=jnp.float32)
            + bc2_ref[...]
        )


def _bcast_spec(shape):
    nd = len(shape)
    return pl.BlockSpec(shape, lambda i: (0,) * nd)


def _edge_phase(xp, att_s, att_d, src, dst, n):
    alpha = att_s[src] + att_d[dst]
    alpha = jnp.where(alpha > 0, alpha, NEG_SLOPE * alpha)
    amax = jax.ops.segment_max(alpha, dst, num_segments=n,
                               indices_are_sorted=True)
    amax = jnp.where(jnp.isfinite(amax), amax, 0.0)
    ex = jnp.exp(alpha - amax[dst])
    denom = jax.ops.segment_sum(ex, dst, num_segments=n,
                                indices_are_sorted=True)
    coef = ex / (denom[dst] + 1e-16)
    xprs = xp.reshape(n, N_HEADS, HEAD_DIM)
    out = jax.ops.segment_sum(xprs[src] * coef[:, :, None], dst,
                              num_segments=n, indices_are_sorted=True)
    return out.reshape(n, N_HEADS * HEAD_DIM)


@functools.partial(jax.jit, static_argnames=())
def kernel(x, edge_index, batch, timestamps, edge_attr, Wt1, bt1, Wt2, bt2,
           Wm1, bm1, Wm2, bm2, Wih, Whh, bih, bhh, g0W, g0as, g0ad, g0b,
           g1W, g1as, g1ad, g1b, Wc1, bc1, Wc2, bc2):
    n = x.shape[0]
    e = edge_index.shape[1]
    mem_dim = Wm2.shape[1]
    src = edge_index[0]
    dst = edge_index[1]

    order = jnp.argsort(dst)
    dst_s = dst[order]
    src_s = src[order]
    ea_s = edge_attr[order]

    ct = jax.lax.stop_gradient(jnp.max(timestamps))
    td = ct.reshape(1, 1)
    te = jnp.maximum(td @ Wt1 + bt1, 0.0) @ Wt2 + bt2

    in_dim = edge_attr.shape[1]
    w1_e = Wm1[2 * mem_dim:2 * mem_dim + in_dim]
    c0 = te @ Wm1[2 * mem_dim + in_dim:] + bm1

    blk_e = 2000
    grid_e = e // blk_e
    msgs = pl.pallas_call(
        _msgs_kernel,
        grid=(grid_e,),
        in_specs=[
            pl.BlockSpec((blk_e, in_dim), lambda i: (i, 0)),
            _bcast_spec(w1_e.shape),
            _bcast_spec((1, mem_dim)),
            _bcast_spec(Wm2.shape),
            _bcast_spec((1, mem_dim)),
        ],
        out_specs=pl.BlockSpec((blk_e, mem_dim), lambda i: (i, 0)),
        out_shape=jax.ShapeDtypeStruct((e, mem_dim), jnp.float32),
    )(ea_s, w1_e, c0, Wm2, bm2.reshape(1, -1))

    ones_e = jnp.ones((e,), jnp.float32)
    counts = jax.ops.segment_sum(ones_e, dst_s, num_segments=n,
                                 indices_are_sorted=True)
    sums = jax.ops.segment_sum(msgs, dst_s, num_segments=n,
                               indices_are_sorted=True)
    agg = sums / jnp.maximum(counts, 1.0)[:, None]

    head_oh = (
        (jnp.arange(mem_dim)[:, None] // HEAD_DIM)
        == jnp.arange(N_HEADS)[None, :]
    ).astype(jnp.float32)
    A0 = jnp.concatenate(
        [g0as.reshape(-1, 1) * head_oh, g0ad.reshape(-1, 1) * head_oh], axis=1)
    A1 = jnp.concatenate(
        [g1as.reshape(-1, 1) * head_oh, g1ad.reshape(-1, 1) * head_oh], axis=1)

    blk_n = 2000
    grid_n = n // blk_n
    xp0, att0 = pl.pallas_call(
        _node0_kernel,
        grid=(grid_n,),
        in_specs=[
            pl.BlockSpec((blk_n, mem_dim), lambda i: (i, 0)),
            pl.BlockSpec((blk_n, 1), lambda i: (i, 0)),
            _bcast_spec((mem_dim, 3 * mem_dim)),
            _bcast_spec((1, 3 * mem_dim)),
            _bcast_spec((1, 3 * mem_dim)),
            _bcast_spec(g0W.shape),
            _bcast_spec((mem_dim, 2 * N_HEADS)),
        ],
        out_specs=[
            pl.BlockSpec((blk_n, mem_dim), lambda i: (i, 0)),
            pl.BlockSpec((blk_n, 2 * N_HEADS), lambda i: (i, 0)),
        ],
        out_shape=[
            jax.ShapeDtypeStruct((n, mem_dim), jnp.float32),
            jax.ShapeDtypeStruct((n, 2 * N_HEADS), jnp.float32),
        ],
    )(agg, counts[:, None], Wih.T, bih.reshape(1, -1), bhh.reshape(1, -1),
      g0W, A0)

    loops = jnp.arange(n, dtype=edge_index.dtype)
    dst_c = jnp.concatenate([dst_s, loops])
    src_c = jnp.concatenate([src_s, loops])
    order_f = jnp.argsort(dst_c)
    dst_f = dst_c[order_f]
    src_f = src_c[order_f]

    out0 = _edge_phase(xp0, att0[:, :N_HEADS], att0[:, N_HEADS:], src_f,
                       dst_f, n)

    xp1, att1 = pl.pallas_call(
        _node1_kernel,
        grid=(grid_n,),
        in_specs=[
            pl.BlockSpec((blk_n, mem_dim), lambda i: (i, 0)),
            _bcast_spec((1, mem_dim)),
            _bcast_spec(g1W.shape),
            _bcast_spec((mem_dim, 2 * N_HEADS)),
        ],
        out_specs=[
            pl.BlockSpec((blk_n, mem_dim), lambda i: (i, 0)),
            pl.BlockSpec((blk_n, 2 * N_HEADS), lambda i: (i, 0)),
        ],
        out_shape=[
            jax.ShapeDtypeStruct((n, mem_dim), jnp.float32),
            jax.ShapeDtypeStruct((n, 2 * N_HEADS), jnp.float32),
        ],
    )(out0, g0b.reshape(1, -1), g1W, A1)

    out1 = _edge_phase(xp1, att1[:, :N_HEADS], att1[:, N_HEADS:], src_f,
                       dst_f, n)

    hid_half = Wc1.shape[1]
    out_dim = Wc2.shape[1]
    wc2_pad = jnp.zeros((hid_half, mem_dim), jnp.float32).at[:, :out_dim].set(Wc2)
    bc2_pad = jnp.zeros((1, mem_dim), jnp.float32).at[0, :out_dim].set(bc2)
    inv_n = jnp.full((1, 1), 1.0 / n, jnp.float32)

    res = pl.pallas_call(
        _pool_kernel,
        grid=(grid_n,),
        in_specs=[
            pl.BlockSpec((blk_n, mem_dim), lambda i: (i, 0)),
            _bcast_spec((1, mem_dim)),
            _bcast_spec(Wc1.shape),
            _bcast_spec((1, hid_half)),
            _bcast_spec(wc2_pad.shape),
            _bcast_spec((1, mem_dim)),
            _bcast_spec((1, 1)),
        ],
        out_specs=pl.BlockSpec((1, mem_dim), lambda i: (0, 0)),
        out_shape=jax.ShapeDtypeStruct((1, mem_dim), jnp.float32),
        scratch_shapes=[pltpu.VMEM((8, mem_dim), jnp.float32)],
    )(out1, g1b.reshape(1, -1), Wc1, bc1.reshape(1, -1), wc2_pad, bc2_pad,
      inv_n)

    return res[:, :out_dim]


# speedup vs baseline: 6.6933x; 6.6933x over previous
import functools

import jax
import jax.numpy as jnp
from jax.experimental import pallas as pl
from jax.experimental.pallas import tpu as pltpu

N_HEADS = 4
HEAD_DIM = 32
NEG_SLOPE = 0.2


def _msgs_kernel(ea_ref, w1_ref, c0_ref, w2_ref, b2_ref, out_ref):
    h = jnp.dot(ea_ref[...], w1_ref[...], preferred_element_type=jnp.float32)
    h = jnp.maximum(h + c0_ref[...], 0.0)
    out_ref[...] = (
        jnp.dot(h, w2_ref[...], preferred_element_type=jnp.float32) + b2_ref[...]
    )


def _node0_kernel(agg_ref, cnt_ref, wihT_ref, bih_ref, bhh_ref, gw_ref, a_ref,
                  xp_ref, att_ref):
    gi = jnp.dot(agg_ref[...], wihT_ref[...], preferred_element_type=jnp.float32)
    gi = gi + bih_ref[...]
    bhh = bhh_ref[...]
    r = jax.nn.sigmoid(gi[:, :128] + bhh[:, :128])
    z = jax.nn.sigmoid(gi[:, 128:256] + bhh[:, 128:256])
    cand = jnp.tanh(gi[:, 256:] + r * bhh[:, 256:])
    mem = (1.0 - z) * cand
    mem = jnp.where(cnt_ref[...] > 0.0, mem, 0.0)
    xp = jnp.dot(mem, gw_ref[...], preferred_element_type=jnp.float32)
    xp_ref[...] = xp
    att_ref[...] = jnp.dot(xp, a_ref[...], preferred_element_type=jnp.float32)


def _node1_kernel(out0_ref, gb_ref, gw_ref, a_ref, xp_ref, att_ref):
    v = out0_ref[...] + gb_ref[...]
    h = jnp.where(v > 0.0, v, jnp.exp(v) - 1.0)
    xp = jnp.dot(h, gw_ref[...], preferred_element_type=jnp.float32)
    xp_ref[...] = xp
    att_ref[...] = jnp.dot(xp, a_ref[...], preferred_element_type=jnp.float32)


def _pool_kernel(out1_ref, gb_ref, wc1_ref, bc1_ref, wc2_ref, bc2_ref, inv_ref,
                 res_ref, acc_ref):
    i = pl.program_id(0)

    @pl.when(i == 0)
    def _init():
        acc_ref[...] = jnp.zeros_like(acc_ref)

    v = out1_ref[...] + gb_ref[...]
    h = jnp.where(v > 0.0, v, jnp.exp(v) - 1.0)
    acc_ref[0:1, :] += jnp.sum(h, axis=0, keepdims=True)

    @pl.when(i == pl.num_programs(0) - 1)
    def _final():
        pooled = acc_ref[0:1, :] * inv_ref[...]
        t = jnp.maximum(
            jnp.dot(pooled, wc1_ref[...], preferred_element_type=jnp.float32)
            + bc1_ref[...],
            0.0,
        )
        res_ref[...] = (
            jnp.dot(t, wc2_ref[...], preferred_element_type=jnp.float32)
            + bc2_ref[...]
        )


def _bcast_spec(shape):
    nd = len(shape)
    return pl.BlockSpec(shape, lambda i: (0,) * nd)


def _edge_phase(xp, att_s, att_d, src, dst, n):
    alpha = att_s[src] + att_d[dst]
    alpha = jnp.where(alpha > 0, alpha, NEG_SLOPE * alpha)
    ex = jnp.exp(alpha - jax.lax.stop_gradient(jnp.max(alpha)))
    xprs = xp.reshape(n, N_HEADS, HEAD_DIM)
    num = (xprs[src] * ex[:, :, None]).reshape(-1, N_HEADS * HEAD_DIM)
    vals = jnp.concatenate([num, ex], axis=1)
    seg = jax.ops.segment_sum(vals, dst, num_segments=n)
    denom = seg[:, N_HEADS * HEAD_DIM:]
    out = seg[:, :N_HEADS * HEAD_DIM].reshape(n, N_HEADS, HEAD_DIM) / (
        denom[:, :, None] + 1e-16)
    return out.reshape(n, N_HEADS * HEAD_DIM)


@functools.partial(jax.jit, static_argnames=())
def kernel(x, edge_index, batch, timestamps, edge_attr, Wt1, bt1, Wt2, bt2,
           Wm1, bm1, Wm2, bm2, Wih, Whh, bih, bhh, g0W, g0as, g0ad, g0b,
           g1W, g1as, g1ad, g1b, Wc1, bc1, Wc2, bc2):
    n = x.shape[0]
    e = edge_index.shape[1]
    mem_dim = Wm2.shape[1]
    src = edge_index[0]
    dst = edge_index[1]

    ct = jax.lax.stop_gradient(jnp.max(timestamps))
    td = ct.reshape(1, 1)
    te = jnp.maximum(td @ Wt1 + bt1, 0.0) @ Wt2 + bt2

    in_dim = edge_attr.shape[1]
    w1_e = Wm1[2 * mem_dim:2 * mem_dim + in_dim]
    c0 = te @ Wm1[2 * mem_dim + in_dim:] + bm1

    blk_e = 2000
    grid_e = e // blk_e
    msgs = pl.pallas_call(
        _msgs_kernel,
        grid=(grid_e,),
        in_specs=[
            pl.BlockSpec((blk_e, in_dim), lambda i: (i, 0)),
            _bcast_spec(w1_e.shape),
            _bcast_spec((1, mem_dim)),
            _bcast_spec(Wm2.shape),
            _bcast_spec((1, mem_dim)),
        ],
        out_specs=pl.BlockSpec((blk_e, mem_dim), lambda i: (i, 0)),
        out_shape=jax.ShapeDtypeStruct((e, mem_dim), jnp.float32),
    )(edge_attr, w1_e, c0, Wm2, bm2.reshape(1, -1))

    ones_e = jnp.ones((e, 1), jnp.float32)
    seg = jax.ops.segment_sum(jnp.concatenate([msgs, ones_e], axis=1), dst,
                              num_segments=n)
    sums = seg[:, :mem_dim]
    counts = seg[:, mem_dim]
    agg = sums / jnp.maximum(counts, 1.0)[:, None]

    head_oh = (
        (jnp.arange(mem_dim)[:, None] // HEAD_DIM)
        == jnp.arange(N_HEADS)[None, :]
    ).astype(jnp.float32)
    A0 = jnp.concatenate(
        [g0as.reshape(-1, 1) * head_oh, g0ad.reshape(-1, 1) * head_oh], axis=1)
    A1 = jnp.concatenate(
        [g1as.reshape(-1, 1) * head_oh, g1ad.reshape(-1, 1) * head_oh], axis=1)

    blk_n = 2000
    grid_n = n // blk_n
    xp0, att0 = pl.pallas_call(
        _node0_kernel,
        grid=(grid_n,),
        in_specs=[
            pl.BlockSpec((blk_n, mem_dim), lambda i: (i, 0)),
            pl.BlockSpec((blk_n, 1), lambda i: (i, 0)),
            _bcast_spec((mem_dim, 3 * mem_dim)),
            _bcast_spec((1, 3 * mem_dim)),
            _bcast_spec((1, 3 * mem_dim)),
            _bcast_spec(g0W.shape),
            _bcast_spec((mem_dim, 2 * N_HEADS)),
        ],
        out_specs=[
            pl.BlockSpec((blk_n, mem_dim), lambda i: (i, 0)),
            pl.BlockSpec((blk_n, 2 * N_HEADS), lambda i: (i, 0)),
        ],
        out_shape=[
            jax.ShapeDtypeStruct((n, mem_dim), jnp.float32),
            jax.ShapeDtypeStruct((n, 2 * N_HEADS), jnp.float32),
        ],
    )(agg, counts[:, None], Wih.T, bih.reshape(1, -1), bhh.reshape(1, -1),
      g0W, A0)

    loops = jnp.arange(n, dtype=edge_index.dtype)
    src_f = jnp.concatenate([src, loops])
    dst_f = jnp.concatenate([dst, loops])

    out0 = _edge_phase(xp0, att0[:, :N_HEADS], att0[:, N_HEADS:], src_f,
                       dst_f, n)

    xp1, att1 = pl.pallas_call(
        _node1_kernel,
        grid=(grid_n,),
        in_specs=[
            pl.BlockSpec((blk_n, mem_dim), lambda i: (i, 0)),
            _bcast_spec((1, mem_dim)),
            _bcast_spec(g1W.shape),
            _bcast_spec((mem_dim, 2 * N_HEADS)),
        ],
        out_specs=[
            pl.BlockSpec((blk_n, mem_dim), lambda i: (i, 0)),
            pl.BlockSpec((blk_n, 2 * N_HEADS), lambda i: (i, 0)),
        ],
        out_shape=[
            jax.ShapeDtypeStruct((n, mem_dim), jnp.float32),
            jax.ShapeDtypeStruct((n, 2 * N_HEADS), jnp.float32),
        ],
    )(out0, g0b.reshape(1, -1), g1W, A1)

    out1 = _edge_phase(xp1, att1[:, :N_HEADS], att1[:, N_HEADS:], src_f,
                       dst_f, n)

    hid_half = Wc1.shape[1]
    out_dim = Wc2.shape[1]
    wc2_pad = jnp.zeros((hid_half, mem_dim), jnp.float32).at[:, :out_dim].set(Wc2)
    bc2_pad = jnp.zeros((1, mem_dim), jnp.float32).at[0, :out_dim].set(bc2)
    inv_n = jnp.full((1, 1), 1.0 / n, jnp.float32)

    res = pl.pallas_call(
        _pool_kernel,
        grid=(grid_n,),
        in_specs=[
            pl.BlockSpec((blk_n, mem_dim), lambda i: (i, 0)),
            _bcast_spec((1, mem_dim)),
            _bcast_spec(Wc1.shape),
            _bcast_spec((1, hid_half)),
            _bcast_spec(wc2_pad.shape),
            _bcast_spec((1, mem_dim)),
            _bcast_spec((1, 1)),
        ],
        out_specs=pl.BlockSpec((1, mem_dim), lambda i: (0, 0)),
        out_shape=jax.ShapeDtypeStruct((1, mem_dim), jnp.float32),
        scratch_shapes=[pltpu.VMEM((8, mem_dim), jnp.float32)],
    )(out1, g1b.reshape(1, -1), Wc1, bc1.reshape(1, -1), wc2_pad, bc2_pad,
      inv_n)

    return res[:, :out_dim]


# speedup vs baseline: 9.5491x; 1.4267x over previous
import functools

import jax
import jax.numpy as jnp
from jax.experimental import pallas as pl
from jax.experimental.pallas import tpu as pltpu

N_HEADS = 4
HEAD_DIM = 32
NEG_SLOPE = 0.2


def _msgs_kernel(ea_ref, w1_ref, c0_ref, w2_ref, b2_ref, out_ref):
    h = jnp.dot(ea_ref[...], w1_ref[...], preferred_element_type=jnp.float32)
    h = jnp.maximum(h + c0_ref[...], 0.0)
    out_ref[...] = (
        jnp.dot(h, w2_ref[...], preferred_element_type=jnp.float32) + b2_ref[...]
    )


def _node0_kernel(agg_ref, cnt_ref, wihT_ref, bih_ref, bhh_ref, gw_ref, a_ref,
                  xp_ref, att_ref):
    gi = jnp.dot(agg_ref[...], wihT_ref[...], preferred_element_type=jnp.float32)
    gi = gi + bih_ref[...]
    bhh = bhh_ref[...]
    r = jax.nn.sigmoid(gi[:, :128] + bhh[:, :128])
    z = jax.nn.sigmoid(gi[:, 128:256] + bhh[:, 128:256])
    cand = jnp.tanh(gi[:, 256:] + r * bhh[:, 256:])
    mem = (1.0 - z) * cand
    mem = jnp.where(cnt_ref[...] > 0.0, mem, 0.0)
    xp = jnp.dot(mem, gw_ref[...], preferred_element_type=jnp.float32)
    xp_ref[...] = xp
    att_ref[...] = jnp.dot(xp, a_ref[...], preferred_element_type=jnp.float32)


def _node1_kernel(out0_ref, gb_ref, gw_ref, a_ref, xp_ref, att_ref):
    v = out0_ref[...] + gb_ref[...]
    h = jnp.where(v > 0.0, v, jnp.exp(v) - 1.0)
    xp = jnp.dot(h, gw_ref[...], preferred_element_type=jnp.float32)
    xp_ref[...] = xp
    att_ref[...] = jnp.dot(xp, a_ref[...], preferred_element_type=jnp.float32)


def _pool_kernel(out1_ref, gb_ref, wc1_ref, bc1_ref, wc2_ref, bc2_ref, inv_ref,
                 res_ref, acc_ref):
    i = pl.program_id(0)

    @pl.when(i == 0)
    def _init():
        acc_ref[...] = jnp.zeros_like(acc_ref)

    v = out1_ref[...] + gb_ref[...]
    h = jnp.where(v > 0.0, v, jnp.exp(v) - 1.0)
    acc_ref[0:1, :] += jnp.sum(h, axis=0, keepdims=True)

    @pl.when(i == pl.num_programs(0) - 1)
    def _final():
        pooled = acc_ref[0:1, :] * inv_ref[...]
        t = jnp.maximum(
            jnp.dot(pooled, wc1_ref[...], preferred_element_type=jnp.float32)
            + bc1_ref[...],
            0.0,
        )
        res_ref[...] = (
            jnp.dot(t, wc2_ref[...], preferred_element_type=jnp.float32)
            + bc2_ref[...]
        )


def _bcast_spec(shape):
    nd = len(shape)
    return pl.BlockSpec(shape, lambda i: (0,) * nd)


def _edge_vals_kernel(as_ref, ad_ref, xg_ref, shift_ref, hT_ref, out_ref):
    alpha = as_ref[...] + ad_ref[...]
    alpha = jnp.where(alpha > 0, alpha, NEG_SLOPE * alpha)
    ex = jnp.exp(alpha - shift_ref[...])
    exb = jnp.dot(ex, hT_ref[...], preferred_element_type=jnp.float32)
    out_ref[:, :N_HEADS * HEAD_DIM] = xg_ref[...] * exb
    out_ref[:, N_HEADS * HEAD_DIM:] = ex


def _edge_phase(xp, att_s, att_d, src, dst, n, hT):
    as_g = att_s[src]
    ad_g = att_d[dst]
    xg = xp[src]
    shift = (jnp.max(att_s, axis=0) + jnp.max(att_d, axis=0)).reshape(1, -1)
    e_f = src.shape[0]
    blk = 2000
    width = N_HEADS * HEAD_DIM + N_HEADS
    vals = pl.pallas_call(
        _edge_vals_kernel,
        grid=(e_f // blk,),
        in_specs=[
            pl.BlockSpec((blk, N_HEADS), lambda i: (i, 0)),
            pl.BlockSpec((blk, N_HEADS), lambda i: (i, 0)),
            pl.BlockSpec((blk, N_HEADS * HEAD_DIM), lambda i: (i, 0)),
            _bcast_spec((1, N_HEADS)),
            _bcast_spec(hT.shape),
        ],
        out_specs=pl.BlockSpec((blk, width), lambda i: (i, 0)),
        out_shape=jax.ShapeDtypeStruct((e_f, width), jnp.float32),
    )(as_g, ad_g, xg, shift, hT)
    seg = jax.ops.segment_sum(vals, dst, num_segments=n)
    denom = seg[:, N_HEADS * HEAD_DIM:]
    out = seg[:, :N_HEADS * HEAD_DIM].reshape(n, N_HEADS, HEAD_DIM) / (
        denom[:, :, None] + 1e-16)
    return out.reshape(n, N_HEADS * HEAD_DIM)


@functools.partial(jax.jit, static_argnames=())
def kernel(x, edge_index, batch, timestamps, edge_attr, Wt1, bt1, Wt2, bt2,
           Wm1, bm1, Wm2, bm2, Wih, Whh, bih, bhh, g0W, g0as, g0ad, g0b,
           g1W, g1as, g1ad, g1b, Wc1, bc1, Wc2, bc2):
    n = x.shape[0]
    e = edge_index.shape[1]
    mem_dim = Wm2.shape[1]
    src = edge_index[0]
    dst = edge_index[1]

    ct = jax.lax.stop_gradient(jnp.max(timestamps))
    td = ct.reshape(1, 1)
    te = jnp.maximum(td @ Wt1 + bt1, 0.0) @ Wt2 + bt2

    in_dim = edge_attr.shape[1]
    w1_e = Wm1[2 * mem_dim:2 * mem_dim + in_dim]
    c0 = te @ Wm1[2 * mem_dim + in_dim:] + bm1

    blk_e = 2000
    grid_e = e // blk_e
    msgs = pl.pallas_call(
        _msgs_kernel,
        grid=(grid_e,),
        in_specs=[
            pl.BlockSpec((blk_e, in_dim), lambda i: (i, 0)),
            _bcast_spec(w1_e.shape),
            _bcast_spec((1, mem_dim)),
            _bcast_spec(Wm2.shape),
            _bcast_spec((1, mem_dim)),
        ],
        out_specs=pl.BlockSpec((blk_e, mem_dim), lambda i: (i, 0)),
        out_shape=jax.ShapeDtypeStruct((e, mem_dim), jnp.float32),
    )(edge_attr, w1_e, c0, Wm2, bm2.reshape(1, -1))

    ones_e = jnp.ones((e, 1), jnp.float32)
    seg = jax.ops.segment_sum(jnp.concatenate([msgs, ones_e], axis=1), dst,
                              num_segments=n)
    sums = seg[:, :mem_dim]
    counts = seg[:, mem_dim]
    agg = sums / jnp.maximum(counts, 1.0)[:, None]

    head_oh = (
        (jnp.arange(mem_dim)[:, None] // HEAD_DIM)
        == jnp.arange(N_HEADS)[None, :]
    ).astype(jnp.float32)
    A0 = jnp.concatenate(
        [g0as.reshape(-1, 1) * head_oh, g0ad.reshape(-1, 1) * head_oh], axis=1)
    A1 = jnp.concatenate(
        [g1as.reshape(-1, 1) * head_oh, g1ad.reshape(-1, 1) * head_oh], axis=1)

    blk_n = 2000
    grid_n = n // blk_n
    xp0, att0 = pl.pallas_call(
        _node0_kernel,
        grid=(grid_n,),
        in_specs=[
            pl.BlockSpec((blk_n, mem_dim), lambda i: (i, 0)),
            pl.BlockSpec((blk_n, 1), lambda i: (i, 0)),
            _bcast_spec((mem_dim, 3 * mem_dim)),
            _bcast_spec((1, 3 * mem_dim)),
            _bcast_spec((1, 3 * mem_dim)),
            _bcast_spec(g0W.shape),
            _bcast_spec((mem_dim, 2 * N_HEADS)),
        ],
        out_specs=[
            pl.BlockSpec((blk_n, mem_dim), lambda i: (i, 0)),
            pl.BlockSpec((blk_n, 2 * N_HEADS), lambda i: (i, 0)),
        ],
        out_shape=[
            jax.ShapeDtypeStruct((n, mem_dim), jnp.float32),
            jax.ShapeDtypeStruct((n, 2 * N_HEADS), jnp.float32),
        ],
    )(agg, counts[:, None], Wih.T, bih.reshape(1, -1), bhh.reshape(1, -1),
      g0W, A0)

    loops = jnp.arange(n, dtype=edge_index.dtype)
    src_f = jnp.concatenate([src, loops])
    dst_f = jnp.concatenate([dst, loops])

    hT = head_oh.T
    out0 = _edge_phase(xp0, att0[:, :N_HEADS], att0[:, N_HEADS:], src_f,
                       dst_f, n, hT)

    xp1, att1 = pl.pallas_call(
        _node1_kernel,
        grid=(grid_n,),
        in_specs=[
            pl.BlockSpec((blk_n, mem_dim), lambda i: (i, 0)),
            _bcast_spec((1, mem_dim)),
            _bcast_spec(g1W.shape),
            _bcast_spec((mem_dim, 2 * N_HEADS)),
        ],
        out_specs=[
            pl.BlockSpec((blk_n, mem_dim), lambda i: (i, 0)),
            pl.BlockSpec((blk_n, 2 * N_HEADS), lambda i: (i, 0)),
        ],
        out_shape=[
            jax.ShapeDtypeStruct((n, mem_dim), jnp.float32),
            jax.ShapeDtypeStruct((n, 2 * N_HEADS), jnp.float32),
        ],
    )(out0, g0b.reshape(1, -1), g1W, A1)

    out1 = _edge_phase(xp1, att1[:, :N_HEADS], att1[:, N_HEADS:], src_f,
                       dst_f, n, hT)

    hid_half = Wc1.shape[1]
    out_dim = Wc2.shape[1]
    wc2_pad = jnp.zeros((hid_half, mem_dim), jnp.float32).at[:, :out_dim].set(Wc2)
    bc2_pad = jnp.zeros((1, mem_dim), jnp.float32).at[0, :out_dim].set(bc2)
    inv_n = jnp.full((1, 1), 1.0 / n, jnp.float32)

    res = pl.pallas_call(
        _pool_kernel,
        grid=(grid_n,),
        in_specs=[
            pl.BlockSpec((blk_n, mem_dim), lambda i: (i, 0)),
            _bcast_spec((1, mem_dim)),
            _bcast_spec(Wc1.shape),
            _bcast_spec((1, hid_half)),
            _bcast_spec(wc2_pad.shape),
            _bcast_spec((1, mem_dim)),
            _bcast_spec((1, 1)),
        ],
        out_specs=pl.BlockSpec((1, mem_dim), lambda i: (0, 0)),
        out_shape=jax.ShapeDtypeStruct((1, mem_dim), jnp.float32),
        scratch_shapes=[pltpu.VMEM((8, mem_dim), jnp.float32)],
    )(out1, g1b.reshape(1, -1), Wc1, bc1.reshape(1, -1), wc2_pad, bc2_pad,
      inv_n)

    return res[:, :out_dim]
